# baseline (speedup 1.0000x reference)
import functools

import jax
import jax.numpy as jnp
from jax import lax
from jax.experimental import pallas as pl
from jax.experimental.pallas import tpu as pltpu
from jax.experimental.pallas import tpu_sc as plsc

N_NODES = 10000
N_PAD = 10240
N_EDGES = 320000
D_FEAT = 128
HIDDEN = 16

NC = 2
NS = 16
NW = NC * NS
CB = 128
NCH = 80
E_PER_W = NCH * CB
E_TOT = NW * E_PER_W
ROWS_PER_TILE = N_PAD // NS


NBUF = 5
ROUNDS = NCH // NBUF


@functools.lru_cache(maxsize=None)
def _make_seg_sum(with_deg: bool):
  mesh = plsc.VectorSubcoreMesh(core_axis_name="c", subcore_axis_name="s")
  out_type = [jax.ShapeDtypeStruct((NC, N_PAD, HIDDEN), jnp.float32)]
  scratch = ([pltpu.VMEM((NCH, CB), jnp.int32),
              pltpu.VMEM((NCH, CB), jnp.int32)]
             + [pltpu.VMEM((CB, HIDDEN), jnp.float32) for _ in range(NBUF)]
             + [pltpu.SemaphoreType.DMA for _ in range(2 * NBUF)]
             + [pltpu.VMEM_SHARED((N_PAD, HIDDEN), jnp.float32)])
  if with_deg:
    out_type.append(jax.ShapeDtypeStruct((NC, N_PAD, HIDDEN), jnp.float32))
    scratch.append(pltpu.VMEM((CB, HIDDEN), jnp.float32))
    scratch.append(pltpu.VMEM_SHARED((N_PAD, HIDDEN), jnp.float32))

  def body(xw_h, src_h, dst_h, zero_h, *rest):
    if with_deg:
      one_h, out_h, deg_out_h = rest[0], rest[1], rest[2]
      rest = rest[3:]
    else:
      out_h = rest[0]
      rest = rest[1:]
    src_v, dst_v = rest[0], rest[1]
    rows_v = rest[2:2 + NBUF]
    gsem = rest[2 + NBUF:2 + 2 * NBUF]
    ssem = rest[2 + 2 * NBUF:2 + 3 * NBUF]
    agg_sh = rest[2 + 3 * NBUF]
    if with_deg:
      ones_v, deg_sh = rest[3 + 3 * NBUF], rest[4 + 3 * NBUF]

    cid = lax.axis_index("c")
    sid = lax.axis_index("s")
    wid = cid * NS + sid
    r0 = sid * ROWS_PER_TILE
    pltpu.sync_copy(zero_h, agg_sh.at[pl.ds(r0, ROWS_PER_TILE)])
    if with_deg:
      pltpu.sync_copy(zero_h, deg_sh.at[pl.ds(r0, ROWS_PER_TILE)])
      pltpu.sync_copy(one_h, ones_v)
    pltpu.sync_copy(src_h.at[wid], src_v)
    pltpu.sync_copy(dst_h.at[wid], dst_v)
    plsc.subcore_barrier()

    for b in range(NBUF):
      pltpu.async_copy(xw_h.at[src_v.at[b]], rows_v[b], gsem[b])

    def round_(r, carry):
      j = r * NBUF
      for b in range(NBUF):
        c = j + b
        pltpu.make_async_copy(xw_h.at[src_v.at[c]], rows_v[b], gsem[b]).wait()
        pltpu.async_copy(rows_v[b], agg_sh.at[dst_v.at[c]], ssem[b], add=True)
        if with_deg:
          pltpu.async_copy(ones_v, deg_sh.at[dst_v.at[c]], ssem[b], add=True)
        nxt = c + NBUF

        @pl.when(nxt < NCH)
        def _():
          pltpu.make_async_copy(rows_v[b], agg_sh.at[dst_v.at[c]],
                                ssem[b]).wait()
          if with_deg:
            pltpu.make_async_copy(ones_v, deg_sh.at[dst_v.at[c]],
                                  ssem[b]).wait()
          pltpu.async_copy(xw_h.at[src_v.at[nxt]], rows_v[b], gsem[b])

      return carry

    lax.fori_loop(0, ROUNDS, round_, 0)
    for b in range(NBUF):
      pltpu.make_async_copy(rows_v[b], agg_sh.at[dst_v.at[0]], ssem[b]).wait()
      if with_deg:
        pltpu.make_async_copy(ones_v, deg_sh.at[dst_v.at[0]], ssem[b]).wait()
    plsc.subcore_barrier()
    pltpu.sync_copy(agg_sh.at[pl.ds(r0, ROWS_PER_TILE)],
                    out_h.at[cid, pl.ds(r0, ROWS_PER_TILE)])
    if with_deg:
      pltpu.sync_copy(deg_sh.at[pl.ds(r0, ROWS_PER_TILE)],
                      deg_out_h.at[cid, pl.ds(r0, ROWS_PER_TILE)])

  return pl.kernel(body, mesh=mesh, out_type=tuple(out_type),
                   scratch_types=tuple(scratch),
                   compiler_params=pltpu.CompilerParams(
                       use_tc_tiling_on_sc=False))


def _seg_sum_deg(*args):
  return _make_seg_sum(True)(*args)


def _seg_sum(*args):
  return _make_seg_sum(False)(*args)


PACK = 8
N8 = N_PAD // PACK
_RB = 256
_NRB = N8 // _RB


def _mm0_body(x_ref, wr_ref, wo_ref, b_ref, xw_ref, root_ref):
  x = x_ref[...]
  xw_ref[...] = jnp.dot(x, wr_ref[...], preferred_element_type=jnp.float32)
  root_ref[...] = (jnp.dot(x, wo_ref[...], preferred_element_type=jnp.float32)
                   + b_ref[...])


def _mm0(x8, wr8, wo8, b8):
  return pl.pallas_call(
      _mm0_body,
      grid=(_NRB,),
      in_specs=[
          pl.BlockSpec((_RB, PACK * D_FEAT), lambda i: (i, 0)),
          pl.BlockSpec((PACK * D_FEAT, 128), lambda i: (0, 0)),
          pl.BlockSpec((PACK * D_FEAT, 128), lambda i: (0, 0)),
          pl.BlockSpec((1, 128), lambda i: (0, 0)),
      ],
      out_specs=[
          pl.BlockSpec((_RB, 128), lambda i: (i, 0)),
          pl.BlockSpec((_RB, 128), lambda i: (i, 0)),
      ],
      out_shape=[
          jax.ShapeDtypeStruct((N8, 128), jnp.float32),
          jax.ShapeDtypeStruct((N8, 128), jnp.float32),
      ],
  )(x8, wr8, wo8, b8)


def _combine_body(p_ref, d_ref, root_ref, wr_ref, wo_ref, b_ref,
                  xw_ref, root_o_ref):
  p = p_ref[...]
  d = d_ref[...]
  h = (p[0] + p[1]) / jnp.maximum(d[0] + d[1], 1.0) + root_ref[...]
  xw_ref[...] = jnp.dot(h, wr_ref[...], preferred_element_type=jnp.float32)
  root_o_ref[...] = (jnp.dot(h, wo_ref[...], preferred_element_type=jnp.float32)
                     + b_ref[...])


def _combine(p, d, root, wr8, wo8, b8):
  return pl.pallas_call(
      _combine_body,
      grid=(_NRB,),
      in_specs=[
          pl.BlockSpec((NC, _RB, 128), lambda i: (0, i, 0)),
          pl.BlockSpec((NC, _RB, 128), lambda i: (0, i, 0)),
          pl.BlockSpec((_RB, 128), lambda i: (i, 0)),
          pl.BlockSpec((128, 128), lambda i: (0, 0)),
          pl.BlockSpec((128, 128), lambda i: (0, 0)),
          pl.BlockSpec((1, 128), lambda i: (0, 0)),
      ],
      out_specs=[
          pl.BlockSpec((_RB, 128), lambda i: (i, 0)),
          pl.BlockSpec((_RB, 128), lambda i: (i, 0)),
      ],
      out_shape=[
          jax.ShapeDtypeStruct((N8, 128), jnp.float32),
          jax.ShapeDtypeStruct((N8, 128), jnp.float32),
      ],
  )(p, d, root, wr8, wo8, b8)


def _final_body(p_ref, d_ref, root_ref, lw_ref, lb_ref, out_ref):
  p = p_ref[...]
  d = d_ref[...]
  h = (p[0] + p[1]) / jnp.maximum(d[0] + d[1], 1.0) + root_ref[...]
  rows = lax.broadcasted_iota(jnp.int32, h.shape, 0)
  h = jnp.where(rows < N_NODES // PACK, h, 0.0)
  s = h[0:256] + h[256:512] + h[512:768] + h[768:1024] + h[1024:1280]
  k = 256
  while k > 8:
    k //= 2
    s = s[:k] + s[k:]
  g = jnp.sum(s, axis=0, keepdims=True) / N_NODES
  out_ref[...] = (jnp.sum(g * lw_ref[...], axis=1, keepdims=True)
                  + lb_ref[...])


def _final(p, d, root, lw, lb):
  return pl.pallas_call(
      _final_body,
      out_shape=jax.ShapeDtypeStruct((1, 1), jnp.float32),
  )(p, d, root, lw, lb)


def _kron8(w):
  k = w.shape[0]
  return (jnp.eye(PACK, dtype=w.dtype)[:, None, :, None]
          * w[None, :, None, :]).reshape(PACK * k, PACK * w.shape[1])


def kernel(x, edge_index, edge_attr, W_rel0, W_root0, b0, W_rel1, W_root1, b1,
           W_rel2, W_root2, b2, lin_W, lin_b):
  npad_e = E_TOT - N_EDGES
  ei32 = edge_index.astype(jnp.int32)
  src = jnp.concatenate(
      [ei32[0], jnp.zeros((npad_e,), jnp.int32)]).reshape(NW, NCH, CB)
  dst = jnp.concatenate(
      [ei32[1], jnp.full((npad_e,), N_PAD - 1, jnp.int32)]).reshape(NW, NCH, CB)
  zero = jnp.zeros((ROWS_PER_TILE, HIDDEN), jnp.float32)
  one = jnp.ones((CB, HIDDEN), jnp.float32)

  x8 = jnp.pad(x, ((0, N_PAD - N_NODES), (0, 0))).reshape(N8, PACK * D_FEAT)
  p2l = lambda a: a.reshape(a.shape[:-2] + (a.shape[-2] * PACK, HIDDEN))
  l2p = lambda a: a.reshape(a.shape[:-2] + (a.shape[-2] // PACK, 128))

  xw, root = _mm0(x8, _kron8(W_rel0), _kron8(W_root0),
                  jnp.tile(b0, PACK).reshape(1, 128))
  p, deg = _seg_sum_deg(p2l(xw), src, dst, zero, one)
  p, deg = l2p(p), l2p(deg)
  xw, root = _combine(p, deg, root, _kron8(W_rel1), _kron8(W_root1),
                      jnp.tile(b1, PACK).reshape(1, 128))
  (p1,) = _seg_sum(p2l(xw), src, dst, zero)
  xw, root = _combine(l2p(p1), deg, root, _kron8(W_rel2), _kron8(W_root2),
                      jnp.tile(b2, PACK).reshape(1, 128))
  (p2,) = _seg_sum(p2l(xw), src, dst, zero)
  return _final(l2p(p2), deg, root,
                jnp.tile(lin_W[:, 0], PACK).reshape(1, 128),
                lin_b.reshape(1, 1))


# speedup vs baseline: 1.6205x; 1.6205x over previous
import functools

import jax
import jax.numpy as jnp
from jax import lax
from jax.experimental import pallas as pl
from jax.experimental.pallas import tpu as pltpu
from jax.experimental.pallas import tpu_sc as plsc

N_NODES = 10000
N_PAD = 10240
N_EDGES = 320000
D_FEAT = 128
HIDDEN = 16

NC = 2
NS = 16
NW = NC * NS
E_PER_W = N_EDGES // NW
CB = 80
NCH = E_PER_W // CB
ROWS_PER_TILE = N_PAD // NS


NBUF = 5
ROUNDS = NCH // NBUF


@functools.lru_cache(maxsize=None)
def _make_seg_sum(with_deg: bool):
  mesh = plsc.VectorSubcoreMesh(core_axis_name="c", subcore_axis_name="s")
  out_type = [jax.ShapeDtypeStruct((NC, N_PAD, HIDDEN), jnp.float32)]
  scratch = ([pltpu.VMEM((NCH, CB), jnp.int32),
              pltpu.VMEM((NCH, CB), jnp.int32)]
             + [pltpu.VMEM((CB, HIDDEN), jnp.float32) for _ in range(NBUF)]
             + [pltpu.SemaphoreType.DMA for _ in range(2 * NBUF)]
             + [pltpu.VMEM_SHARED((N_PAD, HIDDEN), jnp.float32)])
  if with_deg:
    out_type.append(jax.ShapeDtypeStruct((NC, N_PAD, HIDDEN), jnp.float32))
    scratch.append(pltpu.VMEM((CB, HIDDEN), jnp.float32))
    scratch.append(pltpu.VMEM_SHARED((N_PAD, HIDDEN), jnp.float32))

  def body(xw_h, src_h, dst_h, zero_h, *rest):
    if with_deg:
      one_h, out_h, deg_out_h = rest[0], rest[1], rest[2]
      rest = rest[3:]
    else:
      out_h = rest[0]
      rest = rest[1:]
    src_v, dst_v = rest[0], rest[1]
    rows_v = rest[2:2 + NBUF]
    gsem = rest[2 + NBUF:2 + 2 * NBUF]
    ssem = rest[2 + 2 * NBUF:2 + 3 * NBUF]
    agg_sh = rest[2 + 3 * NBUF]
    if with_deg:
      ones_v, deg_sh = rest[3 + 3 * NBUF], rest[4 + 3 * NBUF]

    cid = lax.axis_index("c")
    sid = lax.axis_index("s")
    wid = cid * NS + sid
    r0 = sid * ROWS_PER_TILE
    pltpu.sync_copy(zero_h, agg_sh.at[pl.ds(r0, ROWS_PER_TILE)])
    if with_deg:
      pltpu.sync_copy(zero_h, deg_sh.at[pl.ds(r0, ROWS_PER_TILE)])
      pltpu.sync_copy(one_h, ones_v)
    pltpu.sync_copy(src_h.at[wid], src_v)
    pltpu.sync_copy(dst_h.at[wid], dst_v)
    plsc.subcore_barrier()

    for b in range(NBUF):
      pltpu.async_copy(xw_h.at[src_v.at[b]], rows_v[b], gsem[b])

    def round_(r, carry):
      j = r * NBUF
      for b in range(NBUF):
        c = j + b
        pltpu.make_async_copy(xw_h.at[src_v.at[c]], rows_v[b], gsem[b]).wait()
        pltpu.async_copy(rows_v[b], agg_sh.at[dst_v.at[c]], ssem[b], add=True)
        if with_deg:
          pltpu.async_copy(ones_v, deg_sh.at[dst_v.at[c]], ssem[b], add=True)
        nxt = c + NBUF

        @pl.when(nxt < NCH)
        def _():
          pltpu.make_async_copy(rows_v[b], agg_sh.at[dst_v.at[c]],
                                ssem[b]).wait()
          if with_deg:
            pltpu.make_async_copy(ones_v, deg_sh.at[dst_v.at[c]],
                                  ssem[b]).wait()
          pltpu.async_copy(xw_h.at[src_v.at[nxt]], rows_v[b], gsem[b])

      return carry

    lax.fori_loop(0, ROUNDS, round_, 0)
    for b in range(NBUF):
      pltpu.make_async_copy(rows_v[b], agg_sh.at[dst_v.at[0]], ssem[b]).wait()
      if with_deg:
        pltpu.make_async_copy(ones_v, deg_sh.at[dst_v.at[0]], ssem[b]).wait()
    plsc.subcore_barrier()
    pltpu.sync_copy(agg_sh.at[pl.ds(r0, ROWS_PER_TILE)],
                    out_h.at[cid, pl.ds(r0, ROWS_PER_TILE)])
    if with_deg:
      pltpu.sync_copy(deg_sh.at[pl.ds(r0, ROWS_PER_TILE)],
                      deg_out_h.at[cid, pl.ds(r0, ROWS_PER_TILE)])

  return pl.kernel(body, mesh=mesh, out_type=tuple(out_type),
                   scratch_types=tuple(scratch),
                   compiler_params=pltpu.CompilerParams(
                       use_tc_tiling_on_sc=False))


def _seg_sum_deg(*args):
  return _make_seg_sum(True)(*args)


def _seg_sum(*args):
  return _make_seg_sum(False)(*args)


PACK = 8
N8 = N_PAD // PACK
_RB = 256
_NRB = N8 // _RB


def _mm0_body(x_ref, wr_ref, wo_ref, b_ref, xw_ref, root_ref):
  x = x_ref[...]
  xw_ref[...] = jnp.dot(x, wr_ref[...], preferred_element_type=jnp.float32)
  root_ref[...] = (jnp.dot(x, wo_ref[...], preferred_element_type=jnp.float32)
                   + b_ref[...])


def _mm0(x8, wr8, wo8, b8):
  return pl.pallas_call(
      _mm0_body,
      grid=(_NRB,),
      in_specs=[
          pl.BlockSpec((_RB, PACK * D_FEAT), lambda i: (i, 0)),
          pl.BlockSpec((PACK * D_FEAT, 128), lambda i: (0, 0)),
          pl.BlockSpec((PACK * D_FEAT, 128), lambda i: (0, 0)),
          pl.BlockSpec((1, 128), lambda i: (0, 0)),
      ],
      out_specs=[
          pl.BlockSpec((_RB, 128), lambda i: (i, 0)),
          pl.BlockSpec((_RB, 128), lambda i: (i, 0)),
      ],
      out_shape=[
          jax.ShapeDtypeStruct((N8, 128), jnp.float32),
          jax.ShapeDtypeStruct((N8, 128), jnp.float32),
      ],
  )(x8, wr8, wo8, b8)


def _combine_body(p_ref, d_ref, root_ref, wr_ref, wo_ref, b_ref,
                  xw_ref, root_o_ref):
  p = p_ref[...]
  d = d_ref[...]
  h = (p[0] + p[1]) / jnp.maximum(d[0] + d[1], 1.0) + root_ref[...]
  xw_ref[...] = jnp.dot(h, wr_ref[...], preferred_element_type=jnp.float32)
  root_o_ref[...] = (jnp.dot(h, wo_ref[...], preferred_element_type=jnp.float32)
                     + b_ref[...])


def _combine(p, d, root, wr8, wo8, b8):
  return pl.pallas_call(
      _combine_body,
      grid=(_NRB,),
      in_specs=[
          pl.BlockSpec((NC, _RB, 128), lambda i: (0, i, 0)),
          pl.BlockSpec((NC, _RB, 128), lambda i: (0, i, 0)),
          pl.BlockSpec((_RB, 128), lambda i: (i, 0)),
          pl.BlockSpec((128, 128), lambda i: (0, 0)),
          pl.BlockSpec((128, 128), lambda i: (0, 0)),
          pl.BlockSpec((1, 128), lambda i: (0, 0)),
      ],
      out_specs=[
          pl.BlockSpec((_RB, 128), lambda i: (i, 0)),
          pl.BlockSpec((_RB, 128), lambda i: (i, 0)),
      ],
      out_shape=[
          jax.ShapeDtypeStruct((N8, 128), jnp.float32),
          jax.ShapeDtypeStruct((N8, 128), jnp.float32),
      ],
  )(p, d, root, wr8, wo8, b8)


def _final_body(p_ref, d_ref, root_ref, lw_ref, lb_ref, out_ref):
  p = p_ref[...]
  d = d_ref[...]
  h = (p[0] + p[1]) / jnp.maximum(d[0] + d[1], 1.0) + root_ref[...]
  rows = lax.broadcasted_iota(jnp.int32, h.shape, 0)
  h = jnp.where(rows < N_NODES // PACK, h, 0.0)
  s = h[0:256] + h[256:512] + h[512:768] + h[768:1024] + h[1024:1280]
  k = 256
  while k > 8:
    k //= 2
    s = s[:k] + s[k:]
  g = jnp.sum(s, axis=0, keepdims=True) / N_NODES
  out_ref[...] = (jnp.sum(g * lw_ref[...], axis=1, keepdims=True)
                  + lb_ref[...])


def _final(p, d, root, lw, lb):
  return pl.pallas_call(
      _final_body,
      out_shape=jax.ShapeDtypeStruct((1, 1), jnp.float32),
  )(p, d, root, lw, lb)


def _kron8(w):
  k = w.shape[0]
  return (jnp.eye(PACK, dtype=w.dtype)[:, None, :, None]
          * w[None, :, None, :]).reshape(PACK * k, PACK * w.shape[1])


def kernel(x, edge_index, edge_attr, W_rel0, W_root0, b0, W_rel1, W_root1, b1,
           W_rel2, W_root2, b2, lin_W, lin_b):
  src = edge_index[0].astype(jnp.int32).reshape(NW, NCH, CB)
  dst = edge_index[1].astype(jnp.int32).reshape(NW, NCH, CB)
  zero = jnp.zeros((ROWS_PER_TILE, HIDDEN), jnp.float32)
  one = jnp.ones((CB, HIDDEN), jnp.float32)

  x8 = jnp.pad(x, ((0, N_PAD - N_NODES), (0, 0))).reshape(N8, PACK * D_FEAT)
  p2l = lambda a: a.reshape(a.shape[:-2] + (a.shape[-2] * PACK, HIDDEN))
  l2p = lambda a: a.reshape(a.shape[:-2] + (a.shape[-2] // PACK, 128))

  xw, root = _mm0(x8, _kron8(W_rel0), _kron8(W_root0),
                  jnp.tile(b0, PACK).reshape(1, 128))
  p, deg = _seg_sum_deg(p2l(xw), src, dst, zero, one)
  p, deg = l2p(p), l2p(deg)
  xw, root = _combine(p, deg, root, _kron8(W_rel1), _kron8(W_root1),
                      jnp.tile(b1, PACK).reshape(1, 128))
  (p1,) = _seg_sum(p2l(xw), src, dst, zero)
  xw, root = _combine(l2p(p1), deg, root, _kron8(W_rel2), _kron8(W_root2),
                      jnp.tile(b2, PACK).reshape(1, 128))
  (p2,) = _seg_sum(p2l(xw), src, dst, zero)
  return _final(l2p(p2), deg, root,
                jnp.tile(lin_W[:, 0], PACK).reshape(1, 128),
                lin_b.reshape(1, 1))
